# baseline (speedup 1.0000x reference)
import functools
import jax
import jax.numpy as jnp
from jax import lax
from jax.experimental import pallas as pl
from jax.experimental.pallas import tpu as pltpu
from jax.experimental.pallas import tpu_sc as plsc

B = 8
N = 5000
NP = 5120
C = 80
K1 = 300
S1 = 384
K2 = 208
S2 = 224
F = C * S1
CONF_T = 0.01
IOU_T = 0.45
ONE_BITS = 0x3F800000


def _select_positions(sbits, k, n_idx):
    G, M = sbits.shape
    R = M // 128

    def vbody(_, carry):
        lo, hi = carry
        mid = (lo + hi + 1) // 2
        cnt = jnp.sum((sbits >= mid).astype(jnp.float32), axis=-1, keepdims=True)
        take = cnt >= float(k)
        return jnp.where(take, mid, lo), jnp.where(take, hi, mid - 1)

    lo0 = jnp.zeros((G, 1), jnp.int32)
    hi0 = jnp.full((G, 1), ONE_BITS, jnp.int32)
    tau, _ = lax.fori_loop(0, 31, vbody, (lo0, hi0))

    cnt_gt = jnp.sum((sbits > tau).astype(jnp.float32), axis=-1, keepdims=True)
    rem = float(k) - cnt_gt
    eq = sbits == tau
    idx = lax.broadcasted_iota(jnp.int32, (G, M), 1)

    def ibody(_, carry):
        lo, hi = carry
        mid = (lo + hi) // 2
        cnt = jnp.sum((eq & (idx < mid)).astype(jnp.float32), axis=-1,
                      keepdims=True)
        ge = cnt >= rem
        return jnp.where(ge, lo, mid + 1), jnp.where(ge, mid, hi)

    lo0 = jnp.zeros((G, 1), jnp.int32)
    hi0 = jnp.full((G, 1), n_idx, jnp.int32)
    _, m = lax.fori_loop(0, 15, ibody, (lo0, hi0))

    cand = (sbits > tau) | (eq & (idx < m))
    cf = cand.astype(jnp.float32).reshape(G * R, 128)
    tri_l = (lax.broadcasted_iota(jnp.int32, (128, 128), 0)
             < lax.broadcasted_iota(jnp.int32, (128, 128), 1)).astype(jnp.float32)
    within = jnp.dot(cf, tri_l, preferred_element_type=jnp.float32)
    rows = jnp.sum(cf.reshape(G, R, 128), axis=-1)
    tri_r = (lax.broadcasted_iota(jnp.int32, (R, R), 0)
             < lax.broadcasted_iota(jnp.int32, (R, R), 1)).astype(jnp.float32)
    carry = jnp.dot(rows, tri_r, preferred_element_type=jnp.float32)
    pos = within.reshape(G, R, 128) + carry[:, :, None]
    pos = pos.reshape(G, M).astype(jnp.int32)
    return jnp.where(cand, pos, -1)


def _tc_a_body(sb_ref, posn_ref):
    posn_ref[0] = _select_positions(sb_ref[0], K1, NP)


def _tc_c_body(sb_ref, posn_ref):
    posn_ref[0] = _select_positions(sb_ref[0], K2, F)


def _rank_and_sort(s, ii, chans):
    G, S = s.shape
    lu = lax.broadcasted_iota(jnp.int32, (G, S, S), 1)
    lt = lax.broadcasted_iota(jnp.int32, (G, S, S), 2)
    su, st = s[:, :, None], s[:, None, :]
    iu, it = ii[:, :, None], ii[:, None, :]
    cmp = (su > st) | ((su == st) & ((iu < it) | ((iu == it) & (lu < lt))))
    rank = jnp.sum(cmp.astype(jnp.float32), axis=1).astype(jnp.int32)
    oh = (rank[:, :, None] == lt).astype(jnp.float32)
    outs = [jnp.sum(oh * v[:, :, None], axis=1) for v in [s] + chans]
    return outs[0], outs[1:]


def _tc_b_body(cs_ref, ci_ref, cb_ref, outs_ref, outb_ref, souti_ref):
    s = cs_ref[0]
    ii = ci_ref[0]
    bt = cb_ref[0]
    iif = ii.astype(jnp.float32)
    ss, sb = _rank_and_sort(s, ii, [iif] + [bt[:, j, :] for j in range(4)])
    souti_ref[0] = sb[0].astype(jnp.int32)
    sb = sb[1:]
    b0, b1, b2, b3 = sb
    y1 = jnp.minimum(b0, b2)
    x1 = jnp.minimum(b1, b3)
    y2 = jnp.maximum(b0, b2)
    x2 = jnp.maximum(b1, b3)
    area = (y2 - y1) * (x2 - x1)
    iy1 = jnp.maximum(y1[:, :, None], y1[:, None, :])
    ix1 = jnp.maximum(x1[:, :, None], x1[:, None, :])
    iy2 = jnp.minimum(y2[:, :, None], y2[:, None, :])
    ix2 = jnp.minimum(x2[:, :, None], x2[:, None, :])
    inter = jnp.maximum(iy2 - iy1, 0.0) * jnp.maximum(ix2 - ix1, 0.0)
    union = area[:, :, None] + area[:, None, :] - inter
    iou = inter / jnp.maximum(union, 1e-9)
    Of = (iou > IOU_T).astype(jnp.float32)
    validf = (ss > CONF_T).astype(jnp.float32)

    G = ss.shape[0]
    TW = 32
    n_tiles = 10
    keep = jnp.zeros_like(ss)
    for tb in range(n_tiles):
        lo = tb * TW
        blk = Of[:, :, lo:lo + TW]
        ext = jnp.sum(keep[:, :, None] * blk, axis=1)
        sq = Of[:, lo:lo + TW, lo:lo + TW]
        vb = validf[:, lo:lo + TW]
        kb = jnp.zeros_like(vb)
        for i in range(TW):
            s_i = (jnp.sum(kb * sq[:, :, i], axis=1, keepdims=True)
                   + ext[:, i:i + 1])
            newk = jnp.where(s_i > 0.0, 0.0, vb[:, i:i + 1])
            oh_i = (lax.broadcasted_iota(jnp.int32, (1, TW), 1)
                    == i).astype(jnp.float32)
            kb = kb + newk * oh_i
        pieces = []
        if lo > 0:
            pieces.append(jnp.zeros((G, lo), jnp.float32))
        pieces.append(kb)
        if S1 - lo - TW > 0:
            pieces.append(jnp.zeros((G, S1 - lo - TW), jnp.float32))
        keep = keep + jnp.concatenate(pieces, axis=1)
    outs_ref[0] = ss * keep
    for j in range(4):
        outb_ref[0, :, j, :] = sb[j] * keep


def _tc_d_body(cs_ref, ci_ref, cb_ref, out_ref):
    b = pl.program_id(0)
    s = cs_ref[0]
    ii = ci_ref[0]
    bt = cb_ref[0]
    iif = ii.astype(jnp.float32)
    ss, sc = _rank_and_sort(s, ii, [iif] + [bt[j][None, :] for j in range(4)])
    si = sc[0].astype(jnp.int32) - b * F
    cls = lax.div(si, S1)
    kept = ss > 0.0
    clsf = jnp.where(kept, cls, 0).astype(jnp.float32) + 1.0
    out_ref[0, 0, :] = clsf[0, :200]
    out_ref[0, 1, :] = ss[0, :200]
    for j in range(4):
        bx = jnp.where(kept, jnp.clip(sc[1 + j], 0.0, 1.0), 0.0)
        out_ref[0, 2 + j, :] = bx[0, :200]


def _sc_compact(posn, vals, oidx, boxes, n_rows, n_src, slots, per_worker,
                rows_per_batch, idx_base, n_chunks=1):
    srows = slots // 128
    mesh = plsc.VectorSubcoreMesh(core_axis_name="c", subcore_axis_name="s")
    n_work = (n_rows + per_worker - 1) // per_worker
    has_o = oidx is not None
    CL = n_src // n_chunks

    scratch = [
        pltpu.VMEM((CL,), jnp.int32),
        pltpu.VMEM((CL,), jnp.float32),
        pltpu.VMEM((srows, 128), jnp.float32),
        pltpu.VMEM((srows, 128), jnp.int32),
        pltpu.VMEM((NP, 4), jnp.float32),
        pltpu.VMEM((4, slots), jnp.float32),
    ]
    if has_o:
        scratch += [pltpu.VMEM((CL,), jnp.int32),
                    pltpu.VMEM((srows, 128), jnp.int32)]

    @functools.partial(
        pl.kernel, mesh=mesh,
        compiler_params=pltpu.CompilerParams(needs_layout_passes=False, use_tc_tiling_on_sc=False),
        out_type=(
            pltpu.HBM((n_rows, srows, 128), jnp.float32),
            pltpu.HBM((n_rows, srows, 128), jnp.int32),
            pltpu.HBM((n_rows, 4, slots), jnp.float32),
        ),
        scratch_types=scratch,
    )
    def k(posn_hbm, vals_hbm, *args):
        if has_o:
            (oidx_hbm, boxes_hbm, ov_hbm, oi_hbm, ob_hbm,
             pos_v, val_v, cs_v, ci_v, box_v, ob4_v, oid_v, co_v) = args
        else:
            (boxes_hbm, ov_hbm, oi_hbm, ob_hbm,
             pos_v, val_v, cs_v, ci_v, box_v, ob4_v) = args
        wid = lax.axis_index("s") * 2 + lax.axis_index("c")

        @pl.when(wid < n_work)
        def _():
            b0 = lax.div(wid * per_worker, rows_per_batch)
            pltpu.sync_copy(boxes_hbm.at[pl.ds(b0 * NP, NP)], box_v)

            def do_row(r, _):
                p = wid * per_worker + r

                @pl.when(p < n_rows)
                def _():
                    zf = jnp.zeros((16,), jnp.float32)
                    zi = jnp.zeros((16,), jnp.int32)
                    for rr in range(srows):
                        for j in range(8):
                            cs_v[rr, pl.ds(j * 16, 16)] = zf
                            ci_v[rr, pl.ds(j * 16, 16)] = zi
                            if has_o:
                                co_v[rr, pl.ds(j * 16, 16)] = zi
                    base = lax.div(p, rows_per_batch) * idx_base
                    box_base = lax.div(p, rows_per_batch) * NP
                    lane = lax.iota(jnp.int32, 16)

                    for ch in range(n_chunks):
                        pltpu.sync_copy(posn_hbm.at[p, pl.ds(ch * CL, CL)],
                                        pos_v)
                        pltpu.sync_copy(vals_hbm.at[p, pl.ds(ch * CL, CL)],
                                        val_v)
                        if has_o:
                            pltpu.sync_copy(oidx_hbm.at[p, pl.ds(ch * CL, CL)],
                                            oid_v)
                        cbase = ch * CL + base

                        def scat(j, _):
                            pv = pos_v[pl.ds(j * 16, 16)]
                            sv = val_v[pl.ds(j * 16, 16)]
                            msk = pv >= 0
                            pvc = jnp.maximum(pv, 0)
                            prow = lax.shift_right_logical(pvc, 7)
                            pcol = pvc & 127
                            iv = lane + (j * 16 + cbase)
                            plsc.store_scatter(cs_v, [prow, pcol], sv,
                                               mask=msk)
                            plsc.store_scatter(ci_v, [prow, pcol], iv,
                                               mask=msk)
                            if has_o:
                                ov = oid_v[pl.ds(j * 16, 16)]
                                plsc.store_scatter(co_v, [prow, pcol], ov,
                                                   mask=msk)
                            return 0

                        lax.fori_loop(0, CL // 16, scat, 0)
                    src_v = co_v if has_o else ci_v
                    for j2 in range(slots // 16):
                        rr, off = j2 // 8, (j2 % 8) * 16
                        gid = src_v[rr, pl.ds(off, 16)]
                        loc = jnp.clip(gid - box_base, 0, NP - 1)
                        for cj in range(4):
                            g = plsc.load_gather(
                                box_v, [loc, jnp.full((16,), cj, jnp.int32)])
                            ob4_v[cj, pl.ds(j2 * 16, 16)] = g
                    pltpu.sync_copy(cs_v, ov_hbm.at[p])
                    pltpu.sync_copy(ci_v, oi_hbm.at[p])
                    pltpu.sync_copy(ob4_v, ob_hbm.at[p])
                return 0

            lax.fori_loop(0, per_worker, do_row, 0)

    if has_o:
        return k(posn, vals, oidx, boxes)
    return k(posn, vals, boxes)


def kernel(inputs):
    inputs = inputs.astype(jnp.float32)
    scores = jnp.transpose(inputs[:, :, 1:81], (0, 2, 1))
    scores = jnp.pad(scores, ((0, 0), (0, 0), (0, NP - N)),
                     constant_values=-1.0)
    boxes = inputs[:, :, 81:85]
    boxes_flat = jnp.pad(boxes, ((0, 0), (0, NP - N), (0, 0))
                         ).reshape(B * NP, 4)
    sbits = lax.bitcast_convert_type(scores, jnp.int32)

    posn = pl.pallas_call(
        _tc_a_body,
        grid=(B,),
        in_specs=[pl.BlockSpec((1, C, NP), lambda b: (b, 0, 0))],
        out_specs=pl.BlockSpec((1, C, NP), lambda b: (b, 0, 0)),
        out_shape=jax.ShapeDtypeStruct((B, C, NP), jnp.int32),
    )(sbits)

    cs, ci, cb = _sc_compact(
        posn.reshape(B * C, NP), scores.reshape(B * C, NP), None, boxes_flat,
        n_rows=B * C, n_src=NP, slots=S1, per_worker=20, rows_per_batch=C,
        idx_base=NP)

    cs = cs.reshape(B, C, S1)
    ci = ci.reshape(B, C, S1)
    cbT = cb.reshape(B, C, 4, S1)

    Cc = 8
    outs, outb, souti = pl.pallas_call(
        _tc_b_body,
        grid=(B, C // Cc),
        in_specs=[
            pl.BlockSpec((1, Cc, S1), lambda b, c: (b, c, 0)),
            pl.BlockSpec((1, Cc, S1), lambda b, c: (b, c, 0)),
            pl.BlockSpec((1, Cc, 4, S1), lambda b, c: (b, c, 0, 0)),
        ],
        out_specs=[
            pl.BlockSpec((1, Cc, S1), lambda b, c: (b, c, 0)),
            pl.BlockSpec((1, Cc, 4, S1), lambda b, c: (b, c, 0, 0)),
            pl.BlockSpec((1, Cc, S1), lambda b, c: (b, c, 0)),
        ],
        out_shape=[
            jax.ShapeDtypeStruct((B, C, S1), jnp.float32),
            jax.ShapeDtypeStruct((B, C, 4, S1), jnp.float32),
            jax.ShapeDtypeStruct((B, C, S1), jnp.int32),
        ],
    )(cs, ci, cbT)

    flat_s = outs.reshape(B, F)
    flat_bits = lax.bitcast_convert_type(flat_s, jnp.int32)
    posn2 = pl.pallas_call(
        _tc_c_body,
        grid=(B,),
        in_specs=[pl.BlockSpec((1, 1, F), lambda b: (b, 0, 0))],
        out_specs=pl.BlockSpec((1, 1, F), lambda b: (b, 0, 0)),
        out_shape=jax.ShapeDtypeStruct((B, 1, F), jnp.int32),
    )(flat_bits.reshape(B, 1, F)).reshape(B, F)

    cs2, ci2, cb2 = _sc_compact(
        posn2, flat_s, souti.reshape(B, F), boxes_flat,
        n_rows=B, n_src=F, slots=S2 + 32, per_worker=1, rows_per_batch=1,
        idx_base=F, n_chunks=4)

    cs2 = cs2.reshape(B, S2 + 32)[:, :S2]
    ci2 = ci2.reshape(B, S2 + 32)[:, :S2]
    cb2T = cb2[:, :, :S2]

    out = pl.pallas_call(
        _tc_d_body,
        grid=(B,),
        in_specs=[
            pl.BlockSpec((1, 1, S2), lambda b: (b, 0, 0)),
            pl.BlockSpec((1, 1, S2), lambda b: (b, 0, 0)),
            pl.BlockSpec((1, 4, S2), lambda b: (b, 0, 0)),
        ],
        out_specs=pl.BlockSpec((1, 6, 200), lambda b: (b, 0, 0)),
        out_shape=jax.ShapeDtypeStruct((B, 6, 200), jnp.float32),
    )(cs2.reshape(B, 1, S2), ci2.reshape(B, 1, S2), cb2T)
    return jnp.transpose(out, (0, 2, 1))


# speedup vs baseline: 1.0526x; 1.0526x over previous
import functools
import jax
import jax.numpy as jnp
from jax import lax
from jax.experimental import pallas as pl
from jax.experimental.pallas import tpu as pltpu
from jax.experimental.pallas import tpu_sc as plsc

B = 8
N = 5000
NP = 5120
C = 80
K1 = 300
S1 = 384
K2 = 208
S2 = 224
F = C * S1
CONF_T = 0.01
IOU_T = 0.45
ONE_BITS = 0x3F800000


def _select_positions(sbits, k, n_idx):
    G, M = sbits.shape
    R = M // 128

    def vbody(_, carry):
        lo, hi = carry
        mid = (lo + hi + 1) // 2
        cnt = jnp.sum((sbits >= mid).astype(jnp.float32), axis=-1, keepdims=True)
        take = cnt >= float(k)
        return jnp.where(take, mid, lo), jnp.where(take, hi, mid - 1)

    lo0 = jnp.zeros((G, 1), jnp.int32)
    hi0 = jnp.full((G, 1), ONE_BITS, jnp.int32)
    tau, _ = lax.fori_loop(0, 31, vbody, (lo0, hi0))

    cnt_gt = jnp.sum((sbits > tau).astype(jnp.float32), axis=-1, keepdims=True)
    rem = float(k) - cnt_gt
    eq = sbits == tau
    idx = lax.broadcasted_iota(jnp.int32, (G, M), 1)

    def ibody(_, carry):
        lo, hi = carry
        mid = (lo + hi) // 2
        cnt = jnp.sum((eq & (idx < mid)).astype(jnp.float32), axis=-1,
                      keepdims=True)
        ge = cnt >= rem
        return jnp.where(ge, lo, mid + 1), jnp.where(ge, mid, hi)

    lo0 = jnp.zeros((G, 1), jnp.int32)
    hi0 = jnp.full((G, 1), n_idx, jnp.int32)
    _, m = lax.fori_loop(0, 15, ibody, (lo0, hi0))

    cand = (sbits > tau) | (eq & (idx < m))
    cf = cand.astype(jnp.float32).reshape(G * R, 128)
    tri_l = (lax.broadcasted_iota(jnp.int32, (128, 128), 0)
             < lax.broadcasted_iota(jnp.int32, (128, 128), 1)).astype(jnp.float32)
    within = jnp.dot(cf, tri_l, preferred_element_type=jnp.float32)
    rows = jnp.sum(cf.reshape(G, R, 128), axis=-1)
    tri_r = (lax.broadcasted_iota(jnp.int32, (R, R), 0)
             < lax.broadcasted_iota(jnp.int32, (R, R), 1)).astype(jnp.float32)
    carry = jnp.dot(rows, tri_r, preferred_element_type=jnp.float32)
    pos = within.reshape(G, R, 128) + carry[:, :, None]
    pos = pos.reshape(G, M).astype(jnp.int32)
    return jnp.where(cand, pos, -1)


def _tc_a_body(sb_ref, posn_ref):
    posn_ref[0] = _select_positions(sb_ref[0], K1, NP)


def _tc_c_body(sb_ref, posn_ref):
    posn_ref[0] = _select_positions(sb_ref[0], K2, F)


def _rank_and_sort(s, ii, chans):
    G, S = s.shape
    lt = lax.broadcasted_iota(jnp.int32, (G, S, S), 2)
    lu = None
    su, st = s[:, :, None], s[:, None, :]
    iu, it = ii[:, :, None], ii[:, None, :]
    del lu
    cmp = (su > st) | ((su == st) & (iu < it))
    rank = jnp.sum(cmp.astype(jnp.float32), axis=1).astype(jnp.int32)
    oh = (rank[:, :, None] == lt).astype(jnp.float32)
    outs = [jnp.sum(oh * v[:, :, None], axis=1) for v in [s] + chans]
    return outs[0], outs[1:]


def _tc_b_body(cs_ref, ci_ref, cb_ref, outs_ref, outb_ref, souti_ref,
               of_ref, va_ref):
    s = cs_ref[0]
    ii = ci_ref[0]
    bt = cb_ref[0]
    iif = ii.astype(jnp.float32)
    ss, sb = _rank_and_sort(s, ii, [iif] + [bt[:, j, :] for j in range(4)])
    souti_ref[0] = sb[0].astype(jnp.int32)
    sb = sb[1:]
    b0, b1, b2, b3 = sb
    y1 = jnp.minimum(b0, b2)
    x1 = jnp.minimum(b1, b3)
    y2 = jnp.maximum(b0, b2)
    x2 = jnp.maximum(b1, b3)
    area = (y2 - y1) * (x2 - x1)
    iy1 = jnp.maximum(y1[:, :, None], y1[:, None, :])
    ix1 = jnp.maximum(x1[:, :, None], x1[:, None, :])
    iy2 = jnp.minimum(y2[:, :, None], y2[:, None, :])
    ix2 = jnp.minimum(x2[:, :, None], x2[:, None, :])
    inter = jnp.maximum(iy2 - iy1, 0.0) * jnp.maximum(ix2 - ix1, 0.0)
    union = area[:, :, None] + area[:, None, :] - inter
    iou = inter / jnp.maximum(union, 1e-9)
    of_ref[...] = (iou > IOU_T).astype(jnp.float32)
    validf = (ss > CONF_T).astype(jnp.float32)
    va_ref[...] = jnp.broadcast_to(validf[:, :, None], validf.shape + (128,))
    lane = lax.broadcasted_iota(jnp.int32, ss.shape, 1)

    def body(t, keep):
        row = of_ref[:, pl.ds(t, 1), :][:, 0, :]
        sup = jnp.sum(keep * row, axis=1, keepdims=True)
        v_t = va_ref[:, pl.ds(t, 1), :][:, 0, :1]
        newk = jnp.where(sup > 0.0, 0.0, v_t)
        return keep + newk * (lane == t).astype(jnp.float32)

    keep = lax.fori_loop(0, K1 + 4, body, jnp.zeros_like(ss))
    outs_ref[0] = ss * keep
    for j in range(4):
        outb_ref[0, :, j, :] = sb[j] * keep


def _tc_d_body(cs_ref, ci_ref, cb_ref, out_ref):
    b = pl.program_id(0)
    s = cs_ref[0]
    ii = ci_ref[0]
    bt = cb_ref[0]
    iif = ii.astype(jnp.float32)
    ss, sc = _rank_and_sort(s, ii, [iif] + [bt[j][None, :] for j in range(4)])
    si = sc[0].astype(jnp.int32) - b * F
    cls = lax.div(si, S1)
    kept = ss > 0.0
    clsf = jnp.where(kept, cls, 0).astype(jnp.float32) + 1.0
    out_ref[0, 0, :] = clsf[0, :200]
    out_ref[0, 1, :] = ss[0, :200]
    for j in range(4):
        bx = jnp.where(kept, jnp.clip(sc[1 + j], 0.0, 1.0), 0.0)
        out_ref[0, 2 + j, :] = bx[0, :200]


def _sc_compact(posn, vals, oidx, boxes, n_rows, n_src, slots, per_worker,
                rows_per_batch, idx_base, n_chunks=1):
    srows = slots // 128
    mesh = plsc.VectorSubcoreMesh(core_axis_name="c", subcore_axis_name="s")
    n_work = (n_rows + per_worker - 1) // per_worker
    has_o = oidx is not None
    CL = n_src // n_chunks

    scratch = [
        pltpu.VMEM((CL,), jnp.int32),
        pltpu.VMEM((CL,), jnp.float32),
        pltpu.VMEM((srows, 128), jnp.float32),
        pltpu.VMEM((srows, 128), jnp.int32),
        pltpu.VMEM((NP, 4), jnp.float32),
        pltpu.VMEM((4, slots), jnp.float32),
    ]
    if has_o:
        scratch += [pltpu.VMEM((CL,), jnp.int32),
                    pltpu.VMEM((srows, 128), jnp.int32)]

    @functools.partial(
        pl.kernel, mesh=mesh,
        compiler_params=pltpu.CompilerParams(needs_layout_passes=False, use_tc_tiling_on_sc=False),
        out_type=(
            pltpu.HBM((n_rows, srows, 128), jnp.float32),
            pltpu.HBM((n_rows, srows, 128), jnp.int32),
            pltpu.HBM((n_rows, 4, slots), jnp.float32),
        ),
        scratch_types=scratch,
    )
    def k(posn_hbm, vals_hbm, *args):
        if has_o:
            (oidx_hbm, boxes_hbm, ov_hbm, oi_hbm, ob_hbm,
             pos_v, val_v, cs_v, ci_v, box_v, ob4_v, oid_v, co_v) = args
        else:
            (boxes_hbm, ov_hbm, oi_hbm, ob_hbm,
             pos_v, val_v, cs_v, ci_v, box_v, ob4_v) = args
        wid = lax.axis_index("s") * 2 + lax.axis_index("c")

        @pl.when(wid < n_work)
        def _():
            b0 = lax.div(wid * per_worker, rows_per_batch)
            pltpu.sync_copy(boxes_hbm.at[pl.ds(b0 * NP, NP)], box_v)

            def do_row(r, _):
                p = wid * per_worker + r

                @pl.when(p < n_rows)
                def _():
                    zf = jnp.zeros((16,), jnp.float32)
                    zi = jnp.zeros((16,), jnp.int32)
                    pad_lane = lax.iota(jnp.int32, 16)
                    for rr in range(srows):
                        for j in range(8):
                            cs_v[rr, pl.ds(j * 16, 16)] = zf
                            ci_v[rr, pl.ds(j * 16, 16)] = (
                                pad_lane + (4194304 + rr * 128 + j * 16))
                            if has_o:
                                co_v[rr, pl.ds(j * 16, 16)] = zi
                    base = lax.div(p, rows_per_batch) * idx_base
                    box_base = lax.div(p, rows_per_batch) * NP
                    lane = lax.iota(jnp.int32, 16)

                    for ch in range(n_chunks):
                        pltpu.sync_copy(posn_hbm.at[p, pl.ds(ch * CL, CL)],
                                        pos_v)
                        pltpu.sync_copy(vals_hbm.at[p, pl.ds(ch * CL, CL)],
                                        val_v)
                        if has_o:
                            pltpu.sync_copy(oidx_hbm.at[p, pl.ds(ch * CL, CL)],
                                            oid_v)
                        cbase = ch * CL + base

                        def scat(j, _):
                            pv = pos_v[pl.ds(j * 16, 16)]
                            sv = val_v[pl.ds(j * 16, 16)]
                            msk = pv >= 0
                            pvc = jnp.maximum(pv, 0)
                            prow = lax.shift_right_logical(pvc, 7)
                            pcol = pvc & 127
                            iv = lane + (j * 16 + cbase)
                            plsc.store_scatter(cs_v, [prow, pcol], sv,
                                               mask=msk)
                            plsc.store_scatter(ci_v, [prow, pcol], iv,
                                               mask=msk)
                            if has_o:
                                ov = oid_v[pl.ds(j * 16, 16)]
                                plsc.store_scatter(co_v, [prow, pcol], ov,
                                                   mask=msk)
                            return 0

                        lax.fori_loop(0, CL // 16, scat, 0)
                    src_v = co_v if has_o else ci_v
                    for j2 in range(slots // 16):
                        rr, off = j2 // 8, (j2 % 8) * 16
                        gid = src_v[rr, pl.ds(off, 16)]
                        loc = jnp.clip(gid - box_base, 0, NP - 1)
                        for cj in range(4):
                            g = plsc.load_gather(
                                box_v, [loc, jnp.full((16,), cj, jnp.int32)])
                            ob4_v[cj, pl.ds(j2 * 16, 16)] = g
                    pltpu.sync_copy(cs_v, ov_hbm.at[p])
                    pltpu.sync_copy(ci_v, oi_hbm.at[p])
                    pltpu.sync_copy(ob4_v, ob_hbm.at[p])
                return 0

            lax.fori_loop(0, per_worker, do_row, 0)

    if has_o:
        return k(posn, vals, oidx, boxes)
    return k(posn, vals, boxes)


def kernel(inputs):
    inputs = inputs.astype(jnp.float32)
    scores = jnp.transpose(inputs[:, :, 1:81], (0, 2, 1))
    scores = jnp.pad(scores, ((0, 0), (0, 0), (0, NP - N)),
                     constant_values=-1.0)
    boxes = inputs[:, :, 81:85]
    boxes_flat = jnp.pad(boxes, ((0, 0), (0, NP - N), (0, 0))
                         ).reshape(B * NP, 4)
    sbits = lax.bitcast_convert_type(scores, jnp.int32)

    posn = pl.pallas_call(
        _tc_a_body,
        grid=(B,),
        in_specs=[pl.BlockSpec((1, C, NP), lambda b: (b, 0, 0))],
        out_specs=pl.BlockSpec((1, C, NP), lambda b: (b, 0, 0)),
        out_shape=jax.ShapeDtypeStruct((B, C, NP), jnp.int32),
    )(sbits)

    cs, ci, cb = _sc_compact(
        posn.reshape(B * C, NP), scores.reshape(B * C, NP), None, boxes_flat,
        n_rows=B * C, n_src=NP, slots=S1, per_worker=20, rows_per_batch=C,
        idx_base=NP)

    cs = cs.reshape(B, C, S1)
    ci = ci.reshape(B, C, S1)
    cbT = cb.reshape(B, C, 4, S1)

    Cc = 8
    outs, outb, souti = pl.pallas_call(
        _tc_b_body,
        grid=(B, C // Cc),
        in_specs=[
            pl.BlockSpec((1, Cc, S1), lambda b, c: (b, c, 0)),
            pl.BlockSpec((1, Cc, S1), lambda b, c: (b, c, 0)),
            pl.BlockSpec((1, Cc, 4, S1), lambda b, c: (b, c, 0, 0)),
        ],
        out_specs=[
            pl.BlockSpec((1, Cc, S1), lambda b, c: (b, c, 0)),
            pl.BlockSpec((1, Cc, 4, S1), lambda b, c: (b, c, 0, 0)),
            pl.BlockSpec((1, Cc, S1), lambda b, c: (b, c, 0)),
        ],
        out_shape=[
            jax.ShapeDtypeStruct((B, C, S1), jnp.float32),
            jax.ShapeDtypeStruct((B, C, 4, S1), jnp.float32),
            jax.ShapeDtypeStruct((B, C, S1), jnp.int32),
        ],
        scratch_shapes=[
            pltpu.VMEM((Cc, S1, S1), jnp.float32),
            pltpu.VMEM((Cc, S1, 128), jnp.float32),
        ],
    )(cs, ci, cbT)

    flat_s = outs.reshape(B, F)
    flat_bits = lax.bitcast_convert_type(flat_s, jnp.int32)
    posn2 = pl.pallas_call(
        _tc_c_body,
        grid=(B,),
        in_specs=[pl.BlockSpec((1, 1, F), lambda b: (b, 0, 0))],
        out_specs=pl.BlockSpec((1, 1, F), lambda b: (b, 0, 0)),
        out_shape=jax.ShapeDtypeStruct((B, 1, F), jnp.int32),
    )(flat_bits.reshape(B, 1, F)).reshape(B, F)

    cs2, ci2, cb2 = _sc_compact(
        posn2, flat_s, souti.reshape(B, F), boxes_flat,
        n_rows=B, n_src=F, slots=S2 + 32, per_worker=1, rows_per_batch=1,
        idx_base=F, n_chunks=4)

    cs2 = cs2.reshape(B, S2 + 32)[:, :S2]
    ci2 = ci2.reshape(B, S2 + 32)[:, :S2]
    cb2T = cb2[:, :, :S2]

    out = pl.pallas_call(
        _tc_d_body,
        grid=(B,),
        in_specs=[
            pl.BlockSpec((1, 1, S2), lambda b: (b, 0, 0)),
            pl.BlockSpec((1, 1, S2), lambda b: (b, 0, 0)),
            pl.BlockSpec((1, 4, S2), lambda b: (b, 0, 0)),
        ],
        out_specs=pl.BlockSpec((1, 6, 200), lambda b: (b, 0, 0)),
        out_shape=jax.ShapeDtypeStruct((B, 6, 200), jnp.float32),
    )(cs2.reshape(B, 1, S2), ci2.reshape(B, 1, S2), cb2T)
    return jnp.transpose(out, (0, 2, 1))
